# baseline (speedup 1.0000x reference)
import functools

import jax
import jax.numpy as jnp
from jax.experimental import pallas as pl
from jax.experimental.pallas import tpu as pltpu

N = 100000
E = 1600000
H = 128
L = 4
C = 98

NP = 102400
BLK = 1024
W = 144


def _embed_body(nf_ref, wi_ref, bi_ref, out_ref):
    x = nf_ref[...]
    h = jax.nn.relu(jnp.dot(x, wi_ref[...], preferred_element_type=jnp.float32)
                    + bi_ref[...])
    ones_col = (jax.lax.broadcasted_iota(jnp.int32, (BLK, W - H), 1) == 0
                ).astype(jnp.float32)
    out_ref[...] = jnp.concatenate([h, ones_col], axis=1)


def _layer_core(h_ref, agg_ref, cond_ref, convw_ref, convb_ref, gw_ref,
                gb_ref, bw_ref, bb_ref, lng_ref, lnb_ref):
    h = h_ref[:, :H]
    agg = agg_ref[:, :H]
    cnt = jnp.maximum(agg_ref[:, H:H + 1], 1.0)
    mean = agg / cnt
    hn = jnp.dot(h + mean, convw_ref[...], preferred_element_type=jnp.float32)
    hn = hn + convb_ref[...]
    gamma = jnp.dot(cond_ref[...], gw_ref[...],
                    preferred_element_type=jnp.float32)[0:1, :] + gb_ref[...]
    beta = jnp.dot(cond_ref[...], bw_ref[...],
                   preferred_element_type=jnp.float32)[0:1, :] + bb_ref[...]
    hn = gamma * hn + beta
    r = h + jax.nn.relu(hn)
    mu = jnp.mean(r, axis=1, keepdims=True)
    var = jnp.mean((r - mu) ** 2, axis=1, keepdims=True)
    return (r - mu) * jax.lax.rsqrt(var + 1e-5) * lng_ref[...] + lnb_ref[...]


def _layer_body(h_ref, agg_ref, cond_ref, convw_ref, convb_ref, gw_ref,
                gb_ref, bw_ref, bb_ref, lng_ref, lnb_ref, out_ref):
    y = _layer_core(h_ref, agg_ref, cond_ref, convw_ref, convb_ref, gw_ref,
                    gb_ref, bw_ref, bb_ref, lng_ref, lnb_ref)
    ones_col = (jax.lax.broadcasted_iota(jnp.int32, (BLK, W - H), 1) == 0
                ).astype(jnp.float32)
    out_ref[...] = jnp.concatenate([y, ones_col], axis=1)


def _final_body(h_ref, agg_ref, cond_ref, convw_ref, convb_ref, gw_ref,
                gb_ref, bw_ref, bb_ref, lng_ref, lnb_ref, w1_ref, b1_ref,
                w2_ref, b2_ref, out_ref):
    y = _layer_core(h_ref, agg_ref, cond_ref, convw_ref, convb_ref, gw_ref,
                    gb_ref, bw_ref, bb_ref, lng_ref, lnb_ref)
    z = jax.nn.relu(jnp.dot(y, w1_ref[...], preferred_element_type=jnp.float32)
                    + b1_ref[...])
    out_ref[...] = (jnp.dot(z, w2_ref[...], preferred_element_type=jnp.float32)
                    + b2_ref[...])


def _row_spec(width):
    return pl.BlockSpec((BLK, width), lambda i: (i, 0))


def _full_spec(shape):
    nd = len(shape)
    return pl.BlockSpec(shape, lambda i: (0,) * nd)


def kernel(node_features, edge_index, smpl_params, template_emb, batch_index,
           Wi, bi, convW, convb, gW, gb, bW, bb, lng, lnb, W1, b1, W2, b2):
    f32 = jnp.float32
    grid = NP // BLK

    nf = jnp.zeros((NP, 8), f32).at[:N, :3].set(node_features)
    Wi_p = jnp.zeros((8, H), f32).at[:3].set(Wi)
    cond = jnp.concatenate([smpl_params, template_emb], axis=1)
    cond_p = jnp.zeros((8, H), f32).at[0, :C].set(cond[0])
    gW_p = jnp.zeros((L, H, H), f32).at[:, :C].set(gW)
    bW_p = jnp.zeros((L, H, H), f32).at[:, :C].set(bW)
    W2_p = jnp.zeros((64, 8), f32).at[:, :3].set(W2)
    b2_p = jnp.zeros((8,), f32).at[:3].set(b2)
    bi2 = bi.reshape(1, H)
    b1_2 = b1.reshape(1, 64)
    b2_2 = b2_p.reshape(1, 8)

    src = edge_index[0]
    dst = edge_index[1]

    h = pl.pallas_call(
        _embed_body,
        grid=(grid,),
        in_specs=[_row_spec(8), _full_spec((8, H)), _full_spec((1, H))],
        out_specs=_row_spec(W),
        out_shape=jax.ShapeDtypeStruct((NP, W), f32),
    )(nf, Wi_p, bi2)

    layer_specs = [
        _row_spec(W), _row_spec(W), _full_spec((8, H)), _full_spec((H, H)),
        _full_spec((1, H)), _full_spec((H, H)), _full_spec((1, H)),
        _full_spec((H, H)), _full_spec((1, H)), _full_spec((1, H)),
        _full_spec((1, H)),
    ]

    for l in range(L):
        msg = h[src]
        agg = jnp.zeros((NP, W), f32).at[dst].add(msg)

        args = (h, agg, cond_p, convW[l], convb[l].reshape(1, H), gW_p[l],
                gb[l].reshape(1, H), bW_p[l], bb[l].reshape(1, H),
                lng[l].reshape(1, H), lnb[l].reshape(1, H))
        if l < L - 1:
            h = pl.pallas_call(
                _layer_body,
                grid=(grid,),
                in_specs=layer_specs,
                out_specs=_row_spec(W),
                out_shape=jax.ShapeDtypeStruct((NP, W), f32),
            )(*args)
        else:
            out = pl.pallas_call(
                _final_body,
                grid=(grid,),
                in_specs=layer_specs + [
                    _full_spec((H, 64)), _full_spec((1, 64)),
                    _full_spec((64, 8)), _full_spec((1, 8)),
                ],
                out_specs=_row_spec(8),
                out_shape=jax.ShapeDtypeStruct((NP, 8), f32),
            )(*args, W1, b1_2, W2_p, b2_2)

    return out[:N, :3]


# speedup vs baseline: 6.9457x; 6.9457x over previous
import jax
import jax.numpy as jnp
from jax import lax
from jax.experimental import pallas as pl
from jax.experimental.pallas import tpu as pltpu, tpu_sc as plsc

N = 100000
E = 1600000
H = 128
L = 4
C = 98

CHUNK = 8192
SHIFT = 13
NCH = 13
NP = NCH * CHUNK
BLK = 1024
ACCR = CHUNK + 256
CAP = 50176
EPT = E // 32
EB = 2000
FLUSH = 512
FB = 544
BATCH = 128
ZR = 176

_i32 = jnp.int32
_f32 = jnp.float32


def _mesh():
    return plsc.VectorSubcoreMesh(core_axis_name="c", subcore_axis_name="s")


_SC_PARAMS = pltpu.CompilerParams(needs_layout_passes=False)


def _bucket_body(*refs):
    src_in = refs[0]
    dst_in = refs[1]
    srcb = refs[2]
    ldstb = refs[3]
    cntp = refs[4]
    es = refs[5]
    ed = refs[6]
    fsrc = refs[7:7 + NCH]
    fldst = refs[7 + NCH:7 + 2 * NCH]
    vtmp = refs[7 + 2 * NCH]
    smem = refs[8 + 2 * NCH]

    wid = lax.axis_index("s") * 2 + lax.axis_index("c")
    lanes = lax.iota(_i32, 16)
    sent_src = lanes * 8 + wid * 64
    sent_ldst = CHUNK + lanes * 8 + wid * 4

    def refill(c):
        for k in range(FB // 16):
            fsrc[c][pl.ds(k * 16, 16)] = sent_src
            fldst[c][pl.ds(k * 16, 16)] = sent_ldst

    for c in range(NCH):
        smem[c] = 0
        smem[16 + c] = 0
        refill(c)

    def edge_batch(bi, carry):
        off = pl.multiple_of(wid * EPT + bi * EB, 8)
        pltpu.sync_copy(src_in.at[pl.ds(off, EB)], es)
        pltpu.sync_copy(dst_in.at[pl.ds(off, EB)], ed)

        def vec_step(i, carry2):
            s = es[pl.ds(i * 16, 16)]
            d = ed[pl.ds(i * 16, 16)]
            b = d >> SHIFT
            ld = d & (CHUNK - 1)
            for c in range(NCH):
                m = b == c
                mi = jnp.where(m, 1, 0)
                pfx = plsc.cumsum(mi) - mi
                cnt = smem[c]
                dest = jnp.where(m, cnt + pfx, 528 + lanes)
                plsc.store_scatter(fsrc[c], [dest], s)
                plsc.store_scatter(fldst[c], [dest], ld)
                ncnt = cnt + jnp.max(plsc.all_reduce_population_count(m))
                base = (wid * NCH + c) * CAP

                @pl.when(ncnt >= FLUSH)
                def _flush():
                    hoff = smem[16 + c]
                    ho = pl.multiple_of(base + hoff, FLUSH)
                    pltpu.sync_copy(fsrc[c].at[pl.ds(0, FLUSH)],
                                    srcb.at[pl.ds(ho, FLUSH)])
                    pltpu.sync_copy(fldst[c].at[pl.ds(0, FLUSH)],
                                    ldstb.at[pl.ds(ho, FLUSH)])
                    ts = fsrc[c][pl.ds(FLUSH, 16)]
                    td = fldst[c][pl.ds(FLUSH, 16)]
                    refill(c)
                    fsrc[c][pl.ds(0, 16)] = ts
                    fldst[c][pl.ds(0, 16)] = td
                    smem[16 + c] = hoff + FLUSH
                    smem[c] = ncnt - FLUSH

                @pl.when(ncnt < FLUSH)
                def _keep():
                    smem[c] = ncnt
            return carry2

        lax.fori_loop(0, EB // 16, vec_step, 0)
        return carry

    lax.fori_loop(0, EPT // EB, edge_batch, 0)

    vtmp[pl.ds(0, 16)] = lanes * 0
    for c in range(NCH):
        cnt = smem[c]
        hoff = smem[16 + c]
        base = (wid * NCH + c) * CAP
        ho = pl.multiple_of(base + hoff, FLUSH)
        pltpu.sync_copy(fsrc[c].at[pl.ds(0, FLUSH)],
                        srcb.at[pl.ds(ho, FLUSH)])
        pltpu.sync_copy(fldst[c].at[pl.ds(0, FLUSH)],
                        ldstb.at[pl.ds(ho, FLUSH)])
        total = hoff + ((cnt + BATCH - 1) & ~(BATCH - 1))
        dest = jnp.where(lanes == c, c, 16 + lanes)
        plsc.store_scatter(vtmp, [dest], total + lanes * 0)

    pltpu.sync_copy(vtmp.at[pl.ds(0, 16)],
                    cntp.at[pl.ds(pl.multiple_of(wid * 16, 16), 16)])


def _agg_body(h, srcb, ldstb, cntp, zer_h, agg, acc, rows, isrc, ildst,
              vcnt, gsem, ssem):
    cid = lax.axis_index("c")
    sid = lax.axis_index("s")
    lanes = lax.iota(_i32, 16)

    for chl in range(7):
        ch = cid * 7 + chl

        @pl.when(ch < NCH)
        def _do_chunk():
            for z in range(528 // ZR):
                pltpu.sync_copy(zer_h, acc.at[pl.ds(
                    pl.multiple_of(sid * 528 + z * ZR, 16), ZR)])
            plsc.subcore_barrier()
            for sti in range(2):
                st = sid + sti * 16
                pltpu.sync_copy(
                    cntp.at[pl.ds(pl.multiple_of(st * 16, 16), 16)], vcnt)
                cv = vcnt[pl.ds(0, 16)]
                nb = jnp.max(jnp.where(lanes == ch, cv, 0)) // BATCH
                lbase = (st * NCH + ch) * CAP

                def group(g, carry):
                    pos = pl.multiple_of(lbase + g * 16 * BATCH, BATCH)
                    descs = []
                    for j in range(16):
                        descs.append(pltpu.async_copy(
                            srcb.at[pl.ds(
                                pl.multiple_of(pos + j * BATCH, BATCH),
                                BATCH)],
                            isrc.at[j], gsem))
                        descs.append(pltpu.async_copy(
                            ldstb.at[pl.ds(
                                pl.multiple_of(pos + j * BATCH, BATCH),
                                BATCH)],
                            ildst.at[j], gsem))
                    for dsc in descs:
                        dsc.wait()

                    def batch(j, carry2):
                        pltpu.async_copy(h.at[isrc.at[j]], rows,
                                         gsem).wait()
                        pltpu.async_copy(rows, acc.at[ildst.at[j]], ssem,
                                         add=True).wait()
                        return carry2

                    lax.fori_loop(0, jnp.minimum(16, nb - g * 16), batch, 0)
                    return carry

                lax.fori_loop(0, (nb + 15) // 16, group, 0)
            plsc.subcore_barrier()
            base = sid * 512
            for kk in range(4):
                pltpu.sync_copy(
                    acc.at[pl.ds(pl.multiple_of(base + kk * 128, 8), 128)],
                    rows)
                pltpu.sync_copy(rows, agg.at[pl.ds(
                    pl.multiple_of(ch * CHUNK + base + kk * 128, 8), 128)])
            plsc.subcore_barrier()


def _embed_body(nf_ref, wi_ref, bi_ref, out_ref):
    x = nf_ref[...]
    out_ref[...] = jax.nn.relu(
        jnp.dot(x, wi_ref[...], preferred_element_type=_f32) + bi_ref[...])


def _layer_core(h_ref, agg_ref, cnt_ref, cond_ref, convw_ref, convb_ref,
                gw_ref, gb_ref, bw_ref, bb_ref, lng_ref, lnb_ref):
    h = h_ref[...]
    cnt = jnp.maximum(cnt_ref[:, 0:1], 1.0)
    mean = agg_ref[...] / cnt
    hn = jnp.dot(h + mean, convw_ref[...], preferred_element_type=_f32)
    hn = hn + convb_ref[...]
    gamma = jnp.dot(cond_ref[...], gw_ref[...],
                    preferred_element_type=_f32)[0:1, :] + gb_ref[...]
    beta = jnp.dot(cond_ref[...], bw_ref[...],
                   preferred_element_type=_f32)[0:1, :] + bb_ref[...]
    hn = gamma * hn + beta
    r = h + jax.nn.relu(hn)
    mu = jnp.mean(r, axis=1, keepdims=True)
    var = jnp.mean((r - mu) ** 2, axis=1, keepdims=True)
    return (r - mu) * lax.rsqrt(var + 1e-5) * lng_ref[...] + lnb_ref[...]


def _layer_body(h_ref, agg_ref, cnt_ref, cond_ref, convw_ref, convb_ref,
                gw_ref, gb_ref, bw_ref, bb_ref, lng_ref, lnb_ref, out_ref):
    out_ref[...] = _layer_core(h_ref, agg_ref, cnt_ref, cond_ref, convw_ref,
                               convb_ref, gw_ref, gb_ref, bw_ref, bb_ref,
                               lng_ref, lnb_ref)


def _final_body(h_ref, agg_ref, cnt_ref, cond_ref, convw_ref, convb_ref,
                gw_ref, gb_ref, bw_ref, bb_ref, lng_ref, lnb_ref, w1_ref,
                b1_ref, w2_ref, b2_ref, out_ref):
    y = _layer_core(h_ref, agg_ref, cnt_ref, cond_ref, convw_ref, convb_ref,
                    gw_ref, gb_ref, bw_ref, bb_ref, lng_ref, lnb_ref)
    z = jax.nn.relu(jnp.dot(y, w1_ref[...], preferred_element_type=_f32)
                    + b1_ref[...])
    out_ref[...] = (jnp.dot(z, w2_ref[...], preferred_element_type=_f32)
                    + b2_ref[...])


def _row_spec(width):
    return pl.BlockSpec((BLK, width), lambda i: (i, 0))


def _full_spec(shape):
    nd = len(shape)
    return pl.BlockSpec(shape, lambda i: (0,) * nd)


def kernel(node_features, edge_index, smpl_params, template_emb, batch_index,
           Wi, bi, convW, convb, gW, gb, bW, bb, lng, lnb, W1, b1, W2, b2):
    grid = NP // BLK
    sds = jax.ShapeDtypeStruct

    nf = jnp.zeros((NP, 8), _f32).at[:N, :3].set(node_features)
    Wi_p = jnp.zeros((8, H), _f32).at[:3].set(Wi)
    cond = jnp.concatenate([smpl_params, template_emb], axis=1)
    cond_p = jnp.zeros((8, H), _f32).at[0, :C].set(cond[0])
    gW_p = jnp.zeros((L, H, H), _f32).at[:, :C].set(gW)
    bW_p = jnp.zeros((L, H, H), _f32).at[:, :C].set(bW)
    W2_p = jnp.zeros((64, 8), _f32).at[:, :3].set(W2)
    b2_p = jnp.zeros((8,), _f32).at[:3].set(b2)
    bi2 = bi.reshape(1, H)
    b1_2 = b1.reshape(1, 64)
    b2_2 = b2_p.reshape(1, 8)
    zer_h = jnp.zeros((ZR, H), _f32)
    ones_np = jnp.zeros((NP, H), _f32).at[:, 0].set(1.0)

    bucket = pl.kernel(
        _bucket_body,
        out_type=(sds((32 * NCH * CAP,), _i32), sds((32 * NCH * CAP,), _i32),
                  sds((512,), _i32)),
        mesh=_mesh(),
        compiler_params=_SC_PARAMS,
        scratch_types=(
            [pltpu.VMEM((EB,), _i32), pltpu.VMEM((EB,), _i32)]
            + [pltpu.VMEM((FB,), _i32) for _ in range(2 * NCH)]
            + [pltpu.VMEM((32,), _i32), pltpu.SMEM((64,), _i32)]),
    )
    srcb, ldstb, cntp = bucket(edge_index[0], edge_index[1])

    agg_k = pl.kernel(
        _agg_body,
        out_type=sds((NP, H), _f32),
        mesh=_mesh(),
        compiler_params=_SC_PARAMS,
        scratch_types=(
            pltpu.VMEM_SHARED((ACCR, H), _f32),
            pltpu.VMEM((BATCH, H), _f32),
            pltpu.VMEM((16, BATCH), _i32),
            pltpu.VMEM((16, BATCH), _i32),
            pltpu.VMEM((16,), _i32),
            pltpu.SemaphoreType.DMA,
            pltpu.SemaphoreType.DMA,
        ),
    )

    cnt_full = agg_k(ones_np, srcb, ldstb, cntp, zer_h)

    h = pl.pallas_call(
        _embed_body,
        grid=(grid,),
        in_specs=[_row_spec(8), _full_spec((8, H)), _full_spec((1, H))],
        out_specs=_row_spec(H),
        out_shape=sds((NP, H), _f32),
    )(nf, Wi_p, bi2)

    layer_specs = [
        _row_spec(H), _row_spec(H), _row_spec(H), _full_spec((8, H)),
        _full_spec((H, H)), _full_spec((1, H)), _full_spec((H, H)),
        _full_spec((1, H)), _full_spec((H, H)), _full_spec((1, H)),
        _full_spec((1, H)), _full_spec((1, H)),
    ]

    for l in range(L):
        agg = agg_k(h, srcb, ldstb, cntp, zer_h)
        args = (h, agg, cnt_full, cond_p, convW[l], convb[l].reshape(1, H),
                gW_p[l], gb[l].reshape(1, H), bW_p[l], bb[l].reshape(1, H),
                lng[l].reshape(1, H), lnb[l].reshape(1, H))
        if l < L - 1:
            h = pl.pallas_call(
                _layer_body,
                grid=(grid,),
                in_specs=layer_specs,
                out_specs=_row_spec(H),
                out_shape=sds((NP, H), _f32),
            )(*args)
        else:
            out = pl.pallas_call(
                _final_body,
                grid=(grid,),
                in_specs=layer_specs + [
                    _full_spec((H, 64)), _full_spec((1, 64)),
                    _full_spec((64, 8)), _full_spec((1, 8)),
                ],
                out_specs=_row_spec(8),
                out_shape=sds((NP, 8), _f32),
            )(*args, W1, b1_2, W2_p, b2_2)

    return out[:N, :3]


# speedup vs baseline: 8.0955x; 1.1655x over previous
import jax
import jax.numpy as jnp
from jax import lax
from jax.experimental import pallas as pl
from jax.experimental.pallas import tpu as pltpu, tpu_sc as plsc

N = 100000
E = 1600000
H = 128
L = 4
C = 98

CHUNK = 8192
SHIFT = 13
NCH = 13
NP = NCH * CHUNK
BLK = 1024
ACCR = CHUNK + 128
CAP = 50176
EPT = E // 32
EB = 2000
FLUSH = 512
FB = 544
BATCH = 128
ZR = 130

_i32 = jnp.int32
_f32 = jnp.float32


def _mesh():
    return plsc.VectorSubcoreMesh(core_axis_name="c", subcore_axis_name="s")


_SC_PARAMS = pltpu.CompilerParams(needs_layout_passes=False)


def _bucket_body(*refs):
    src_in = refs[0]
    dst_in = refs[1]
    srcb = refs[2]
    ldstb = refs[3]
    cntp = refs[4]
    es = refs[5]
    ed = refs[6]
    fsrc = refs[7:7 + NCH]
    fldst = refs[7 + NCH:7 + 2 * NCH]
    vtmp = refs[7 + 2 * NCH]
    smem = refs[8 + 2 * NCH]

    wid = lax.axis_index("s") * 2 + lax.axis_index("c")
    lanes = lax.iota(_i32, 16)
    sent_src = lanes * 8 + wid * 64
    sent_ldst = CHUNK + lanes * 4 + wid

    def refill(c):
        for k in range(FB // 16):
            fsrc[c][pl.ds(k * 16, 16)] = sent_src
            fldst[c][pl.ds(k * 16, 16)] = sent_ldst

    for c in range(NCH):
        smem[c] = 0
        smem[16 + c] = 0
        refill(c)

    def edge_batch(bi, carry):
        off = pl.multiple_of(wid * EPT + bi * EB, 8)
        pltpu.sync_copy(src_in.at[pl.ds(off, EB)], es)
        pltpu.sync_copy(dst_in.at[pl.ds(off, EB)], ed)

        def vec_step(i, carry2):
            s = es[pl.ds(i * 16, 16)]
            d = ed[pl.ds(i * 16, 16)]
            b = d >> SHIFT
            ld = d & (CHUNK - 1)
            for c in range(NCH):
                m = b == c
                mi = jnp.where(m, 1, 0)
                pfx = plsc.cumsum(mi) - mi
                cnt = smem[c]
                dest = jnp.where(m, cnt + pfx, 528 + lanes)
                plsc.store_scatter(fsrc[c], [dest], s)
                plsc.store_scatter(fldst[c], [dest], ld)
                ncnt = cnt + jnp.max(plsc.all_reduce_population_count(m))
                base = (wid * NCH + c) * CAP

                @pl.when(ncnt >= FLUSH)
                def _flush():
                    hoff = smem[16 + c]
                    ho = pl.multiple_of(base + hoff, FLUSH)
                    pltpu.sync_copy(fsrc[c].at[pl.ds(0, FLUSH)],
                                    srcb.at[pl.ds(ho, FLUSH)])
                    pltpu.sync_copy(fldst[c].at[pl.ds(0, FLUSH)],
                                    ldstb.at[pl.ds(ho, FLUSH)])
                    ts = fsrc[c][pl.ds(FLUSH, 16)]
                    td = fldst[c][pl.ds(FLUSH, 16)]
                    refill(c)
                    fsrc[c][pl.ds(0, 16)] = ts
                    fldst[c][pl.ds(0, 16)] = td
                    smem[16 + c] = hoff + FLUSH
                    smem[c] = ncnt - FLUSH

                @pl.when(ncnt < FLUSH)
                def _keep():
                    smem[c] = ncnt
            return carry2

        lax.fori_loop(0, EB // 16, vec_step, 0)
        return carry

    lax.fori_loop(0, EPT // EB, edge_batch, 0)

    vtmp[pl.ds(0, 16)] = lanes * 0
    for c in range(NCH):
        cnt = smem[c]
        hoff = smem[16 + c]
        base = (wid * NCH + c) * CAP
        ho = pl.multiple_of(base + hoff, FLUSH)
        pltpu.sync_copy(fsrc[c].at[pl.ds(0, FLUSH)],
                        srcb.at[pl.ds(ho, FLUSH)])
        pltpu.sync_copy(fldst[c].at[pl.ds(0, FLUSH)],
                        ldstb.at[pl.ds(ho, FLUSH)])
        total = hoff + ((cnt + BATCH - 1) & ~(BATCH - 1))
        dest = jnp.where(lanes == c, c, 16 + lanes)
        plsc.store_scatter(vtmp, [dest], total + lanes * 0)

    pltpu.sync_copy(vtmp.at[pl.ds(0, 16)],
                    cntp.at[pl.ds(pl.multiple_of(wid * 16, 16), 16)])


def _make_agg_body(width):
    def body(h, srcb, ldstb, cntp, zer_h, agg, acc, rows, isrc, ildst,
             vcnt, *sems):
        gsems = sems[0:2]
        ssems = sems[2:4]
        isem = sems[4]
        cid = lax.axis_index("c")
        sid = lax.axis_index("s")
        lanes = lax.iota(_i32, 16)

        def chunk_step(chl, carry0):
            ch = cid * 7 + chl

            @pl.when(ch < NCH)
            def _do_chunk():
                for z in range(520 // ZR):
                    pltpu.sync_copy(zer_h, acc.at[pl.ds(
                        pl.multiple_of(sid * 520 + z * ZR, 2), ZR)])
                plsc.subcore_barrier()
                for sti in range(2):
                    st = sid + sti * 16
                    pltpu.sync_copy(
                        cntp.at[pl.ds(pl.multiple_of(st * 16, 16), 16)],
                        vcnt)
                    cv = vcnt[pl.ds(0, 16)]
                    nb = jnp.max(jnp.where(lanes == ch, cv, 0)) // BATCH
                    lbase = (st * NCH + ch) * CAP

                    def group(g, carry):
                        pos = pl.multiple_of(lbase + g * 8 * BATCH, BATCH)
                        jmax = jnp.minimum(8, nb - g * 8)
                        idescs = []
                        for j in range(8):
                            idescs.append(pltpu.async_copy(
                                srcb.at[pl.ds(
                                    pl.multiple_of(pos + j * BATCH, BATCH),
                                    BATCH)],
                                isrc.at[j], isem))
                            idescs.append(pltpu.async_copy(
                                ldstb.at[pl.ds(
                                    pl.multiple_of(pos + j * BATCH, BATCH),
                                    BATCH)],
                                ildst.at[j], isem))
                        for dsc in idescs:
                            dsc.wait()

                        gd = [None] * 8
                        sd = [None] * 8
                        for j in range(8):
                            @pl.when(j < jmax)
                            def _fire(j=j):
                                if j >= 2:
                                    sd[j - 2].wait()
                                gd[j] = pltpu.async_copy(
                                    h.at[isrc.at[j]], rows.at[j % 2],
                                    gsems[j % 2])
                            if j > 0:
                                @pl.when(j - 1 < jmax)
                                def _drain(j=j):
                                    gd[j - 1].wait()
                                    sd[j - 1] = pltpu.async_copy(
                                        rows.at[(j - 1) % 2],
                                        acc.at[ildst.at[j - 1]],
                                        ssems[(j - 1) % 2], add=True)

                        @pl.when(7 < jmax)
                        def _last():
                            gd[7].wait()
                            sd[7] = pltpu.async_copy(
                                rows.at[1], acc.at[ildst.at[7]],
                                ssems[1], add=True)

                        for j in range(8):
                            @pl.when((j >= jmax - 2) & (j < jmax))
                            def _ep(j=j):
                                sd[j].wait()
                        return carry

                    lax.fori_loop(0, (nb + 7) // 8, group, 0)
                plsc.subcore_barrier()
                base = sid * 512
                for kk in range(4):
                    pltpu.sync_copy(
                        acc.at[pl.ds(
                            pl.multiple_of(base + kk * 128, 8), 128)],
                        rows.at[0])
                    pltpu.sync_copy(rows.at[0], agg.at[pl.ds(
                        pl.multiple_of(ch * CHUNK + base + kk * 128, 8),
                        128)])
                plsc.subcore_barrier()
            return carry0

        lax.fori_loop(0, 7, chunk_step, 0)

    return body


def _embed_body(nf_ref, wi_ref, bi_ref, out_ref):
    x = nf_ref[...]
    out_ref[...] = jax.nn.relu(
        jnp.dot(x, wi_ref[...], preferred_element_type=_f32) + bi_ref[...])


def _layer_core(h_ref, agg_ref, cnt_ref, cond_ref, convw_ref, convb_ref,
                gw_ref, gb_ref, bw_ref, bb_ref, lng_ref, lnb_ref):
    h = h_ref[...]
    cnt = jnp.maximum(cnt_ref[:, 0:1], 1.0)
    mean = agg_ref[...] / cnt
    hn = jnp.dot(h + mean, convw_ref[...], preferred_element_type=_f32)
    hn = hn + convb_ref[...]
    gamma = jnp.dot(cond_ref[...], gw_ref[...],
                    preferred_element_type=_f32)[0:1, :] + gb_ref[...]
    beta = jnp.dot(cond_ref[...], bw_ref[...],
                   preferred_element_type=_f32)[0:1, :] + bb_ref[...]
    hn = gamma * hn + beta
    r = h + jax.nn.relu(hn)
    mu = jnp.mean(r, axis=1, keepdims=True)
    var = jnp.mean((r - mu) ** 2, axis=1, keepdims=True)
    return (r - mu) * lax.rsqrt(var + 1e-5) * lng_ref[...] + lnb_ref[...]


def _layer_body(h_ref, agg_ref, cnt_ref, cond_ref, convw_ref, convb_ref,
                gw_ref, gb_ref, bw_ref, bb_ref, lng_ref, lnb_ref, out_ref):
    out_ref[...] = _layer_core(h_ref, agg_ref, cnt_ref, cond_ref, convw_ref,
                               convb_ref, gw_ref, gb_ref, bw_ref, bb_ref,
                               lng_ref, lnb_ref)


def _final_body(h_ref, agg_ref, cnt_ref, cond_ref, convw_ref, convb_ref,
                gw_ref, gb_ref, bw_ref, bb_ref, lng_ref, lnb_ref, w1_ref,
                b1_ref, w2_ref, b2_ref, out_ref):
    y = _layer_core(h_ref, agg_ref, cnt_ref, cond_ref, convw_ref, convb_ref,
                    gw_ref, gb_ref, bw_ref, bb_ref, lng_ref, lnb_ref)
    z = jax.nn.relu(jnp.dot(y, w1_ref[...], preferred_element_type=_f32)
                    + b1_ref[...])
    out_ref[...] = (jnp.dot(z, w2_ref[...], preferred_element_type=_f32)
                    + b2_ref[...])


def _row_spec(width):
    return pl.BlockSpec((BLK, width), lambda i: (i, 0))


def _full_spec(shape):
    nd = len(shape)
    return pl.BlockSpec(shape, lambda i: (0,) * nd)


def kernel(node_features, edge_index, smpl_params, template_emb, batch_index,
           Wi, bi, convW, convb, gW, gb, bW, bb, lng, lnb, W1, b1, W2, b2):
    grid = NP // BLK
    sds = jax.ShapeDtypeStruct

    nf = jnp.zeros((NP, 8), _f32).at[:N, :3].set(node_features)
    Wi_p = jnp.zeros((8, H), _f32).at[:3].set(Wi)
    cond = jnp.concatenate([smpl_params, template_emb], axis=1)
    cond_p = jnp.zeros((8, H), _f32).at[0, :C].set(cond[0])
    gW_p = jnp.zeros((L, H, H), _f32).at[:, :C].set(gW)
    bW_p = jnp.zeros((L, H, H), _f32).at[:, :C].set(bW)
    W2_p = jnp.zeros((64, 8), _f32).at[:, :3].set(W2)
    b2_p = jnp.zeros((8,), _f32).at[:3].set(b2)
    bi2 = bi.reshape(1, H)
    b1_2 = b1.reshape(1, 64)
    b2_2 = b2_p.reshape(1, 8)
    zer_h = jnp.zeros((ZR, H), _f32)
    ones_np = jnp.zeros((NP, H), _f32).at[:, 0].set(1.0)

    bucket = pl.kernel(
        _bucket_body,
        out_type=(sds((32 * NCH * CAP,), _i32), sds((32 * NCH * CAP,), _i32),
                  sds((512,), _i32)),
        mesh=_mesh(),
        compiler_params=_SC_PARAMS,
        scratch_types=(
            [pltpu.VMEM((EB,), _i32), pltpu.VMEM((EB,), _i32)]
            + [pltpu.VMEM((FB,), _i32) for _ in range(2 * NCH)]
            + [pltpu.VMEM((32,), _i32), pltpu.SMEM((64,), _i32)]),
    )
    srcb, ldstb, cntp = bucket(edge_index[0], edge_index[1])

    def make_agg(width):
        return pl.kernel(
            _make_agg_body(width),
            out_type=sds((NP, width), _f32),
            mesh=_mesh(),
            compiler_params=_SC_PARAMS,
            scratch_types=(
                pltpu.VMEM_SHARED((ACCR, width), _f32),
                pltpu.VMEM((2, BATCH, width), _f32),
                pltpu.VMEM((8, BATCH), _i32),
                pltpu.VMEM((8, BATCH), _i32),
                pltpu.VMEM((16,), _i32),
            ) + tuple(pltpu.SemaphoreType.DMA for _ in range(5)),
        )

    agg_k = make_agg(H)

    cnt_full = agg_k(ones_np, srcb, ldstb, cntp, zer_h)

    nf = nf + cnt_full[:, :1] * 0.0

    h = pl.pallas_call(
        _embed_body,
        grid=(grid,),
        in_specs=[_row_spec(8), _full_spec((8, H)), _full_spec((1, H))],
        out_specs=_row_spec(H),
        out_shape=sds((NP, H), _f32),
    )(nf, Wi_p, bi2)

    layer_specs = [
        _row_spec(H), _row_spec(H), _row_spec(H), _full_spec((8, H)),
        _full_spec((H, H)), _full_spec((1, H)), _full_spec((H, H)),
        _full_spec((1, H)), _full_spec((H, H)), _full_spec((1, H)),
        _full_spec((1, H)), _full_spec((1, H)),
    ]

    for l in range(L):
        agg = agg_k(h, srcb, ldstb, cntp, zer_h)
        args = (h, agg, cnt_full, cond_p, convW[l], convb[l].reshape(1, H),
                gW_p[l], gb[l].reshape(1, H), bW_p[l], bb[l].reshape(1, H),
                lng[l].reshape(1, H), lnb[l].reshape(1, H))
        if l < L - 1:
            h = pl.pallas_call(
                _layer_body,
                grid=(grid,),
                in_specs=layer_specs,
                out_specs=_row_spec(H),
                out_shape=sds((NP, H), _f32),
            )(*args)
        else:
            out = pl.pallas_call(
                _final_body,
                grid=(grid,),
                in_specs=layer_specs + [
                    _full_spec((H, 64)), _full_spec((1, 64)),
                    _full_spec((64, 8)), _full_spec((1, 8)),
                ],
                out_specs=_row_spec(8),
                out_shape=sds((NP, 8), _f32),
            )(*args, W1, b1_2, W2_p, b2_2)

    return out[:N, :3]
